# Initial kernel scaffold; baseline (speedup 1.0000x reference)
import jax
import jax.numpy as jnp
from jax.experimental import pallas as pl


def kernel(src, row, col, Height, Width):
    raise NotImplementedError("write your pallas kernel here")


# speedup vs baseline: 1.6699x; 1.6699x over previous
import functools

import jax
import jax.numpy as jnp
from jax import lax
from jax.experimental import pallas as pl
from jax.experimental.pallas import tpu as pltpu
from jax.experimental.pallas import tpu_sc as plsc

T, C, H, W = 64, 256, 56, 56
CH = 8
NCHUNK = C // CH
NWORKERS = 32
CROPS = ((0, 8, 16), (2, 12, 24), (1, 12, 24))


def _build():
    mesh = plsc.VectorSubcoreMesh(core_axis_name="c", subcore_axis_name="s")
    out_type = (
        jax.ShapeDtypeStruct((T, C, 16 * 16), jnp.float32),
        jax.ShapeDtypeStruct((T, C, 24 * 24), jnp.float32),
        jax.ShapeDtypeStruct((T, C, 24 * 24), jnp.float32),
    )
    scratch_types = [
        pltpu.VMEM((CH * H * W,), jnp.float32),
        pltpu.VMEM((CH, 16 * 16), jnp.float32),
        pltpu.VMEM((CH, 24 * 24), jnp.float32),
        pltpu.VMEM((CH, 24 * 24), jnp.float32),
        pltpu.VMEM((T * 7,), jnp.float32),
        pltpu.VMEM((T * 7,), jnp.float32),
    ]

    @functools.partial(
        pl.kernel, mesh=mesh, out_type=out_type,
        scratch_types=scratch_types,
        compiler_params=pltpu.CompilerParams(needs_layout_passes=False))
    def run(src, row, col, face_o, l_o, r_o,
            stage, face_v, l_v, r_v, row_v, col_v):
        wid = lax.axis_index("s") * 2 + lax.axis_index("c")
        pltpu.sync_copy(row, row_v)
        pltpu.sync_copy(col, col_v)
        outs = (face_o, l_o, r_o)
        out_vs = (face_v, l_v, r_v)

        def start_scalar(ref_v, t, kk, half):
            iv = jnp.full((16,), t * 7 + kk, jnp.int32)
            vals = plsc.load_gather(ref_v, [iv])
            s16 = (vals * float(H)).astype(jnp.int32) - half
            return lax.reduce_max(s16, (0,))

        def do_t(ti, carry):
            t = wid + ti * NWORKERS
            starts = []
            for (kk, half, size) in CROPS:
                rs = start_scalar(row_v, t, kk, half)
                cs = start_scalar(col_v, t, kk, half)
                starts.append((rs, cs, size))

            def do_chunk(ci, carry2):
                ch0 = ci * CH
                pltpu.sync_copy(src.at[t, pl.ds(ch0 * H * W, CH * H * W)],
                                stage)
                for oi, (rs, cs, size) in enumerate(starts):
                    nvec = size * size // 16

                    def do_ch(ch, carry3, oi=oi, rs=rs, cs=cs, size=size,
                              nvec=nvec):
                        ov2 = out_vs[oi]
                        chbase = jnp.full((16,), ch * (H * W), jnp.int32)
                        lane = lax.iota(jnp.int32, 16)
                        for v in range(nvec):
                            pos = lane + (v * 16)
                            ivals = pos // size
                            jvals = pos - ivals * size
                            li = rs + ivals
                            sc = cs + jvals
                            valid = ((li >= 0) & (li < H)
                                     & (sc >= 0) & (sc < W))
                            lic = jnp.clip(li, 0, H - 1)
                            scc = jnp.clip(sc, 0, W - 1)
                            idx = chbase + lic * W + scc
                            g = plsc.load_gather(stage, [idx])
                            ov2[ch, pl.ds(v * 16, 16)] = jnp.where(
                                valid, g, 0.0)
                        return carry3

                    lax.fori_loop(0, CH, do_ch, 0)
                for oi2 in range(3):
                    pltpu.sync_copy(out_vs[oi2],
                                    outs[oi2].at[t, pl.ds(ch0, CH)])
                return carry2

            lax.fori_loop(0, NCHUNK, do_chunk, 0)
            return carry

        lax.fori_loop(0, T // NWORKERS, do_t, 0)

    return run


_sc_crop = _build()


def kernel(src, row, col, Height, Width):
    face, l_out, r_out = _sc_crop(src.reshape(T, C * H * W),
                                  row.reshape(T * 7),
                                  col.reshape(T * 7))
    return (face.reshape(T, C, 16, 16),
            l_out.reshape(T, C, 24, 24),
            r_out.reshape(T, C, 24, 24))
